# baseline (speedup 1.0000x reference)
import functools

import jax
import jax.numpy as jnp
from jax.experimental import pallas as pl
from jax.experimental.pallas import tpu as pltpu

_NEGPOS = 3
_VAR0 = 0.1
_VAR1 = 0.2
_THR = 0.35
_L = 128


def _mbl_body(truths_ref, loc_ref, conf_ref, pri_ref, out_ref, nv_ref,
              bov_s, bidx_s, *, P, R, O):
    b = pl.program_id(0)
    L = _L
    NC = R // 8

    iota8 = (jax.lax.broadcasted_iota(jnp.int32, (8, L), 0) * L
             + jax.lax.broadcasted_iota(jnp.int32, (8, L), 1))

    txs = []
    for t in range(O):
        txs.append((truths_ref[b, 4 * t + 0],
                    truths_ref[b, 4 * t + 1],
                    truths_ref[b, 4 * t + 2],
                    truths_ref[b, 4 * t + 3]))

    vmax = [jnp.full((8, L), -1.0, jnp.float32) for _ in range(O)]
    vidx = [jnp.zeros((8, L), jnp.int32) for _ in range(O)]
    for c in range(NC):
        s = c * 8
        cx = pri_ref[0, s:s + 8, :]
        cy = pri_ref[1, s:s + 8, :]
        w = pri_ref[2, s:s + 8, :]
        h = pri_ref[3, s:s + 8, :]
        p1 = cx - w / 2
        q1 = cy - h / 2
        p2 = cx + w / 2
        q2 = cy + h / 2
        ap = (p2 - p1) * (q2 - q1)
        flatc = iota8 + c * (8 * L)
        bov = jnp.full((8, L), -1.0, jnp.float32)
        bidx = jnp.zeros((8, L), jnp.int32)
        for t in range(O):
            tx1, ty1, tx2, ty2 = txs[t]
            ix = jnp.maximum(jnp.minimum(tx2, p2) - jnp.maximum(tx1, p1),
                             0.0)
            iy = jnp.maximum(jnp.minimum(ty2, q2) - jnp.maximum(ty1, q1),
                             0.0)
            inter = ix * iy
            area_t = (tx2 - tx1) * (ty2 - ty1)
            iou = inter / (area_t + ap - inter)
            u = iou > bov
            bidx = jnp.where(u, t, bidx)
            bov = jnp.where(u, iou, bov)
            u2 = iou > vmax[t]
            vidx[t] = jnp.where(u2, flatc, vidx[t])
            vmax[t] = jnp.where(u2, iou, vmax[t])
        bov_s[s:s + 8, :] = bov
        bidx_s[s:s + 8, :] = bidx

    bps = []
    for t in range(O):
        m = jnp.max(vmax[t])
        bps.append(jnp.min(jnp.where(vmax[t] == m, vidx[t],
                                     jnp.int32(0x7FFFFFFF))))

    row_i = jax.lax.broadcasted_iota(jnp.int32, (R, L), 0)
    lane_i = jax.lax.broadcasted_iota(jnp.int32, (R, L), 1)
    flat = row_i * L + lane_i

    best_ov = bov_s[...]
    best_idx = bidx_s[...]
    for t in range(O):
        hit = flat == bps[t]
        best_ov = jnp.where(hit, 2.0, best_ov)
        best_idx = jnp.where(hit, t, best_idx)

    pos = best_ov >= _THR
    np_lp = jnp.sum(pos.astype(jnp.float32), axis=0, keepdims=True)

    mx1 = jnp.zeros((R, L), jnp.float32)
    my1 = jnp.zeros((R, L), jnp.float32)
    mx2 = jnp.zeros((R, L), jnp.float32)
    my2 = jnp.zeros((R, L), jnp.float32)
    for t in range(O):
        sel = best_idx == t
        tx1, ty1, tx2, ty2 = txs[t]
        mx1 = jnp.where(sel, tx1, mx1)
        my1 = jnp.where(sel, ty1, my1)
        mx2 = jnp.where(sel, tx2, mx2)
        my2 = jnp.where(sel, ty2, my2)

    pcx = pri_ref[0]
    pcy = pri_ref[1]
    pw = pri_ref[2]
    ph = pri_ref[3]
    g_cx = ((mx1 + mx2) / 2 - pcx) / (_VAR0 * pw)
    g_cy = ((my1 + my2) / 2 - pcy) / (_VAR0 * ph)
    g_w = jnp.log((mx2 - mx1) / pw) / _VAR1
    g_h = jnp.log((my2 - my1) / ph) / _VAR1

    def sl1(pred, g):
        ad = jnp.abs(pred - g)
        return jnp.where(ad < 1.0, 0.5 * ad * ad, ad - 0.5)

    q4 = jax.lax.broadcasted_iota(jnp.int32, (4 * L, L), 0)
    p4 = jax.lax.broadcasted_iota(jnp.int32, (4 * L, L), 1)
    locI = loc_ref[0]
    locs = []
    for c_ in range(4):
        wsel = (q4 == 4 * p4 + c_).astype(jnp.float32)
        locs.append(jnp.dot(locI, wsel,
                            preferred_element_type=jnp.float32))
    q2 = jax.lax.broadcasted_iota(jnp.int32, (2 * L, L), 0)
    p2 = jax.lax.broadcasted_iota(jnp.int32, (2 * L, L), 1)
    confI = conf_ref[0]
    confs = []
    for c_ in range(2):
        wsel = (q2 == 2 * p2 + c_).astype(jnp.float32)
        confs.append(jnp.dot(confI, wsel,
                             preferred_element_type=jnp.float32))

    lsum = (sl1(locs[0], g_cx) + sl1(locs[1], g_cy)
            + sl1(locs[2], g_w) + sl1(locs[3], g_h))
    ll_lp = jnp.sum(jnp.where(pos, lsum, 0.0), axis=0, keepdims=True)

    c0 = confs[0]
    c1 = confs[1]
    cm = jnp.maximum(c0, c1)
    lse = cm + jnp.log(jnp.exp(c0 - cm) + jnp.exp(c1 - cm))
    ce = lse - jnp.where(pos, c1, c0)
    pc_lp = jnp.sum(jnp.where(pos, ce, 0.0), axis=0, keepdims=True)

    nv_ref[0] = jnp.where(pos, -1.0, ce)

    r8 = jax.lax.broadcasted_iota(jnp.int32, (8, L), 0)
    o = jnp.zeros((8, L), jnp.float32)
    o = jnp.where(r8 == 0, np_lp, o)
    o = jnp.where(r8 == 1, ll_lp, o)
    o = jnp.where(r8 == 2, pc_lp, o)
    out_ref[0] = o


def _sel_body(o1_ref, nv_ref, out_ref, *, P, B):
    L = _L
    o1 = o1_ref[...]
    np_v = jnp.sum(o1[:, 0:1, :], axis=(1, 2), keepdims=True)
    ll_v = jnp.sum(o1[:, 1:2, :], axis=(1, 2), keepdims=True)
    pc_v = jnp.sum(o1[:, 2:3, :], axis=(1, 2), keepdims=True)
    np_i = np_v.astype(jnp.int32)
    nv = nv_ref[...]
    nvi = jax.lax.bitcast_convert_type(nv, jnp.int32)
    k = jnp.minimum(_NEGPOS * np_i, P - 1)
    k = jnp.minimum(k, P - np_i)
    thr = jnp.zeros((B, 1, 1), jnp.int32)
    for bit in range(30, -1, -1):
        cand = thr | jnp.int32(1 << bit)
        cnt = jnp.sum((nvi >= cand).astype(jnp.int32), axis=(1, 2),
                      keepdims=True)
        thr = jnp.where(cnt >= k, cand, thr)
    cnt_gt = jnp.sum((nvi > thr).astype(jnp.int32), axis=(1, 2),
                     keepdims=True)
    sum_gt = jnp.sum(jnp.where(nvi > thr, nv, 0.0), axis=(1, 2),
                     keepdims=True)
    thr_f = jax.lax.bitcast_convert_type(thr, jnp.float32)
    neg_sum = jnp.where(
        k > 0, sum_gt + (k - cnt_gt).astype(jnp.float32) * thr_f, 0.0)
    n_total = jnp.maximum(jnp.sum(np_v), 1.0)
    loss_l = jnp.sum(ll_v) / n_total
    loss_c = (jnp.sum(pc_v) + jnp.sum(neg_sum)) / n_total
    r8 = jax.lax.broadcasted_iota(jnp.int32, (8, L), 0)
    l8 = jax.lax.broadcasted_iota(jnp.int32, (8, L), 1)
    o = jnp.zeros((8, L), jnp.float32)
    o = jnp.where((r8 == 0) & (l8 == 0), loss_l, o)
    o = jnp.where((r8 == 0) & (l8 == 1), loss_c, o)
    out_ref[...] = o


@jax.jit
def kernel(loc_data, conf_data, priors, targets):
    B, P, _ = loc_data.shape
    O = targets.shape[1]
    L = _L
    PP = ((P + 8 * L - 1) // (8 * L)) * (8 * L)
    R = PP // L
    pad = PP - P

    locp = jnp.pad(loc_data, ((0, 0), (0, pad), (0, 0)))
    locp = locp.reshape(B, R, 4 * L)
    conf_pad = jnp.broadcast_to(jnp.array([100.0, -100.0], jnp.float32),
                                (B, pad, 2))
    confp = jnp.concatenate([conf_data, conf_pad], axis=1)
    confp = confp.reshape(B, R, 2 * L)
    pad_pri = jnp.tile(jnp.array([[3.0, 3.0, 1.0, 1.0]], jnp.float32),
                       (pad, 1))
    prip = jnp.concatenate([priors, pad_pri], axis=0).T.reshape(4, R, L)
    truths2 = targets[..., :4].reshape(B, 4 * O)

    out1, nvv = pl.pallas_call(
        functools.partial(_mbl_body, P=P, R=R, O=O),
        grid=(B,),
        in_specs=[
            pl.BlockSpec(memory_space=pltpu.SMEM),
            pl.BlockSpec((1, R, 4 * L), lambda b: (b, 0, 0)),
            pl.BlockSpec((1, R, 2 * L), lambda b: (b, 0, 0)),
            pl.BlockSpec((4, R, L), lambda b: (0, 0, 0)),
        ],
        out_specs=[
            pl.BlockSpec((1, 8, L), lambda b: (b, 0, 0)),
            pl.BlockSpec((1, R, L), lambda b: (b, 0, 0)),
        ],
        out_shape=[
            jax.ShapeDtypeStruct((B, 8, L), jnp.float32),
            jax.ShapeDtypeStruct((B, R, L), jnp.float32),
        ],
        scratch_shapes=[
            pltpu.VMEM((R, L), jnp.float32),
            pltpu.VMEM((R, L), jnp.int32),
        ],
        compiler_params=pltpu.CompilerParams(
            dimension_semantics=("arbitrary",)),
    )(truths2, locp, confp, prip)

    out2 = pl.pallas_call(
        functools.partial(_sel_body, P=P, B=B),
        out_shape=jax.ShapeDtypeStruct((8, L), jnp.float32),
    )(out1, nvv)
    return out2[0, 0], out2[0, 1]


# speedup vs baseline: 1.1671x; 1.1671x over previous
import functools

import jax
import jax.numpy as jnp
from jax.experimental import pallas as pl
from jax.experimental.pallas import tpu as pltpu

_NEGPOS = 3
_VAR0 = 0.1
_VAR1 = 0.2
_THR = 0.35
_L = 128


def _mbl_body(truths_ref, loc_ref, conf_ref, pri_ref, out_ref, nv_ref,
              bov_s, bidx_s, *, P, R, O):
    b = pl.program_id(0)
    L = _L
    NC = R // 8

    iota8 = (jax.lax.broadcasted_iota(jnp.int32, (8, L), 0) * L
             + jax.lax.broadcasted_iota(jnp.int32, (8, L), 1))

    txs = []
    for t in range(O):
        txs.append((truths_ref[b, 4 * t + 0],
                    truths_ref[b, 4 * t + 1],
                    truths_ref[b, 4 * t + 2],
                    truths_ref[b, 4 * t + 3]))

    vmax = [jnp.full((8, L), -1.0, jnp.float32) for _ in range(O)]
    vidx = [jnp.zeros((8, L), jnp.int32) for _ in range(O)]
    for c in range(NC):
        s = c * 8
        cx = pri_ref[0, s:s + 8, :]
        cy = pri_ref[1, s:s + 8, :]
        w = pri_ref[2, s:s + 8, :]
        h = pri_ref[3, s:s + 8, :]
        p1 = cx - w / 2
        q1 = cy - h / 2
        p2 = cx + w / 2
        q2 = cy + h / 2
        ap = (p2 - p1) * (q2 - q1)
        flatc = iota8 + c * (8 * L)
        bov = jnp.full((8, L), -1.0, jnp.float32)
        bidx = jnp.zeros((8, L), jnp.int32)
        for t in range(O):
            tx1, ty1, tx2, ty2 = txs[t]
            ix = jnp.maximum(jnp.minimum(tx2, p2) - jnp.maximum(tx1, p1),
                             0.0)
            iy = jnp.maximum(jnp.minimum(ty2, q2) - jnp.maximum(ty1, q1),
                             0.0)
            inter = ix * iy
            area_t = (tx2 - tx1) * (ty2 - ty1)
            iou = inter / (area_t + ap - inter)
            u = iou > bov
            bidx = jnp.where(u, t, bidx)
            bov = jnp.where(u, iou, bov)
            u2 = iou > vmax[t]
            vidx[t] = jnp.where(u2, flatc, vidx[t])
            vmax[t] = jnp.where(u2, iou, vmax[t])
        bov_s[s:s + 8, :] = bov
        bidx_s[s:s + 8, :] = bidx

    bps = []
    for t in range(O):
        m = jnp.max(vmax[t])
        bps.append(jnp.min(jnp.where(vmax[t] == m, vidx[t],
                                     jnp.int32(0x7FFFFFFF))))

    row_i = jax.lax.broadcasted_iota(jnp.int32, (R, L), 0)
    lane_i = jax.lax.broadcasted_iota(jnp.int32, (R, L), 1)
    flat = row_i * L + lane_i

    best_ov = bov_s[...]
    best_idx = bidx_s[...]
    for t in range(O):
        hit = flat == bps[t]
        best_ov = jnp.where(hit, 2.0, best_ov)
        best_idx = jnp.where(hit, t, best_idx)

    pos = best_ov >= _THR
    np_lp = jnp.sum(pos.astype(jnp.float32), axis=0, keepdims=True)

    mx1 = jnp.zeros((R, L), jnp.float32)
    my1 = jnp.zeros((R, L), jnp.float32)
    mx2 = jnp.zeros((R, L), jnp.float32)
    my2 = jnp.zeros((R, L), jnp.float32)
    for t in range(O):
        sel = best_idx == t
        tx1, ty1, tx2, ty2 = txs[t]
        mx1 = jnp.where(sel, tx1, mx1)
        my1 = jnp.where(sel, ty1, my1)
        mx2 = jnp.where(sel, tx2, mx2)
        my2 = jnp.where(sel, ty2, my2)

    pcx = pri_ref[0]
    pcy = pri_ref[1]
    pw = pri_ref[2]
    ph = pri_ref[3]
    g_cx = ((mx1 + mx2) / 2 - pcx) / (_VAR0 * pw)
    g_cy = ((my1 + my2) / 2 - pcy) / (_VAR0 * ph)
    g_w = jnp.log((mx2 - mx1) / pw) / _VAR1
    g_h = jnp.log((my2 - my1) / ph) / _VAR1

    def sl1(pred, g):
        ad = jnp.abs(pred - g)
        return jnp.where(ad < 1.0, 0.5 * ad * ad, ad - 0.5)

    lsum = (sl1(loc_ref[0, 0], g_cx) + sl1(loc_ref[0, 1], g_cy)
            + sl1(loc_ref[0, 2], g_w) + sl1(loc_ref[0, 3], g_h))
    ll_lp = jnp.sum(jnp.where(pos, lsum, 0.0), axis=0, keepdims=True)

    c0 = conf_ref[0, 0]
    c1 = conf_ref[0, 1]
    cm = jnp.maximum(c0, c1)
    lse = cm + jnp.log(jnp.exp(c0 - cm) + jnp.exp(c1 - cm))
    ce = lse - jnp.where(pos, c1, c0)
    pc_lp = jnp.sum(jnp.where(pos, ce, 0.0), axis=0, keepdims=True)

    nv_ref[0] = jnp.where(pos, -1.0, ce)

    r8 = jax.lax.broadcasted_iota(jnp.int32, (8, L), 0)
    o = jnp.zeros((8, L), jnp.float32)
    o = jnp.where(r8 == 0, np_lp, o)
    o = jnp.where(r8 == 1, ll_lp, o)
    o = jnp.where(r8 == 2, pc_lp, o)
    out_ref[0] = o


def _sel_body(o1_ref, nv_ref, out_ref, *, P, B):
    L = _L
    o1 = o1_ref[...]
    np_v = jnp.sum(o1[:, 0:1, :], axis=(1, 2), keepdims=True)
    ll_v = jnp.sum(o1[:, 1:2, :], axis=(1, 2), keepdims=True)
    pc_v = jnp.sum(o1[:, 2:3, :], axis=(1, 2), keepdims=True)
    np_i = np_v.astype(jnp.int32)
    nv = nv_ref[...]
    nvi = jax.lax.bitcast_convert_type(nv, jnp.int32)
    k = jnp.minimum(_NEGPOS * np_i, P - 1)
    k = jnp.minimum(k, P - np_i)
    thr = jnp.zeros((B, 1, 1), jnp.int32)
    for bit in range(30, -1, -1):
        cand = thr | jnp.int32(1 << bit)
        cnt = jnp.sum((nvi >= cand).astype(jnp.int32), axis=(1, 2),
                      keepdims=True)
        thr = jnp.where(cnt >= k, cand, thr)
    cnt_gt = jnp.sum((nvi > thr).astype(jnp.int32), axis=(1, 2),
                     keepdims=True)
    sum_gt = jnp.sum(jnp.where(nvi > thr, nv, 0.0), axis=(1, 2),
                     keepdims=True)
    thr_f = jax.lax.bitcast_convert_type(thr, jnp.float32)
    neg_sum = jnp.where(
        k > 0, sum_gt + (k - cnt_gt).astype(jnp.float32) * thr_f, 0.0)
    n_total = jnp.maximum(jnp.sum(np_v), 1.0)
    loss_l = jnp.sum(ll_v) / n_total
    loss_c = (jnp.sum(pc_v) + jnp.sum(neg_sum)) / n_total
    r8 = jax.lax.broadcasted_iota(jnp.int32, (8, L), 0)
    l8 = jax.lax.broadcasted_iota(jnp.int32, (8, L), 1)
    o = jnp.zeros((8, L), jnp.float32)
    o = jnp.where((r8 == 0) & (l8 == 0), loss_l, o)
    o = jnp.where((r8 == 0) & (l8 == 1), loss_c, o)
    out_ref[...] = o


@jax.jit
def kernel(loc_data, conf_data, priors, targets):
    B, P, _ = loc_data.shape
    O = targets.shape[1]
    L = _L
    PP = ((P + 8 * L - 1) // (8 * L)) * (8 * L)
    R = PP // L
    pad = PP - P

    locp = jnp.pad(loc_data, ((0, 0), (0, pad), (0, 0)))
    locp = locp.reshape(B, R, L, 4).transpose(0, 3, 1, 2)
    conf_pad = jnp.broadcast_to(jnp.array([100.0, -100.0], jnp.float32),
                                (B, pad, 2))
    confp = jnp.concatenate([conf_data, conf_pad], axis=1)
    confp = confp.reshape(B, R, L, 2).transpose(0, 3, 1, 2)
    pad_pri = jnp.tile(jnp.array([[3.0, 3.0, 1.0, 1.0]], jnp.float32),
                       (pad, 1))
    prip = jnp.concatenate([priors, pad_pri], axis=0).T.reshape(4, R, L)
    truths2 = targets[..., :4].reshape(B, 4 * O)

    out1, nvv = pl.pallas_call(
        functools.partial(_mbl_body, P=P, R=R, O=O),
        grid=(B,),
        in_specs=[
            pl.BlockSpec(memory_space=pltpu.SMEM),
            pl.BlockSpec((1, 4, R, L), lambda b: (b, 0, 0, 0)),
            pl.BlockSpec((1, 2, R, L), lambda b: (b, 0, 0, 0)),
            pl.BlockSpec((4, R, L), lambda b: (0, 0, 0)),
        ],
        out_specs=[
            pl.BlockSpec((1, 8, L), lambda b: (b, 0, 0)),
            pl.BlockSpec((1, R, L), lambda b: (b, 0, 0)),
        ],
        out_shape=[
            jax.ShapeDtypeStruct((B, 8, L), jnp.float32),
            jax.ShapeDtypeStruct((B, R, L), jnp.float32),
        ],
        scratch_shapes=[
            pltpu.VMEM((R, L), jnp.float32),
            pltpu.VMEM((R, L), jnp.int32),
        ],
        compiler_params=pltpu.CompilerParams(
            dimension_semantics=("arbitrary",)),
    )(truths2, locp, confp, prip)

    out2 = pl.pallas_call(
        functools.partial(_sel_body, P=P, B=B),
        out_shape=jax.ShapeDtypeStruct((8, L), jnp.float32),
    )(out1, nvv)
    return out2[0, 0], out2[0, 1]
